# baseline (speedup 1.0000x reference)
import functools

import jax
import jax.numpy as jnp
from jax import lax
from jax.experimental import pallas as pl
from jax.experimental.pallas import tpu as pltpu
from jax.experimental.pallas import tpu_sc as plsc

N, C, H, W = 4, 96, 224, 224
P = H * W
R = 3 * N * P
NC, NS, L = 2, 16, 16
NW = NC * NS
RPW = R // NW
B = 128
CP = 128
NCHUNK = RPW // B
GROUPS = B // L

_SUBL = 392
_LANE = 128


def _corners(ix, iy, base, with_mask):
    x0f = jnp.floor(ix)
    y0f = jnp.floor(iy)
    x1f = x0f + 1.0
    y1f = y0f + 1.0
    wx1 = ix - x0f
    wx0 = 1.0 - wx1
    wy1 = iy - y0f
    wy0 = 1.0 - wy1

    def clampi(v, hi):
        return jnp.clip(v, 0.0, hi).astype(jnp.int32)

    xi0 = clampi(x0f, W - 1.0)
    xi1 = clampi(x1f, W - 1.0)
    yi0 = clampi(y0f, H - 1.0)
    yi1 = clampi(y1f, H - 1.0)

    def flat(yi, xi):
        return base + yi * W + xi

    idx = (flat(yi0, xi0), flat(yi0, xi1), flat(yi1, xi0), flat(yi1, xi1))
    w = [wy0 * wx0, wy0 * wx1, wy1 * wx0, wy1 * wx1]
    if with_mask:
        def m(yf, xf):
            return ((xf >= 0.0) & (xf <= W - 1.0)
                    & (yf >= 0.0) & (yf <= H - 1.0)).astype(jnp.float32)
        w[0] = w[0] * m(y0f, x0f)
        w[1] = w[1] * m(y0f, x1f)
        w[2] = w[2] * m(y1f, x0f)
        w[3] = w[3] * m(y1f, x1f)
    return idx, tuple(w)


def _reflect(c, span):
    cc = jnp.abs(c)
    extra = jnp.mod(cc, span)
    flips = jnp.floor(cc / span)
    return jnp.where(jnp.mod(flips, 2.0) == 0.0, extra, span - extra)


def _idx_weight_body(gx_ref, gy_ref, i0, i1, i2, i3, w0, w1, w2, w3):
    n = pl.program_id(0)
    base = n * P
    gx = gx_ref[0]
    gy = gy_ref[0]
    irefs = (i0, i1, i2, i3)
    wrefs = (w0, w1, w2, w3)

    ix = (gx + 1.0) * (0.5 * (W - 1.0))
    iy = (gy + 1.0) * (0.5 * (H - 1.0))
    idx, wgt = _corners(ix, iy, base, with_mask=True)
    for k in range(4):
        irefs[k][0, 0] = idx[k]
        wrefs[k][0, 0] = wgt[k]

    ix = jnp.clip(((gx + 1.0) * W - 1.0) * 0.5, 0.0, W - 1.0)
    iy = jnp.clip(((gy + 1.0) * H - 1.0) * 0.5, 0.0, H - 1.0)
    idx, wgt = _corners(ix, iy, base, with_mask=False)
    for k in range(4):
        irefs[k][1, 0] = idx[k]
        wrefs[k][1, 0] = wgt[k]

    ix = (gx + 1.0) * (0.5 * (W - 1.0))
    iy = (gy + 1.0) * (0.5 * (H - 1.0))
    ix = jnp.clip(_reflect(ix, W - 1.0), 0.0, W - 1.0)
    iy = jnp.clip(_reflect(iy, H - 1.0), 0.0, H - 1.0)
    idx, wgt = _corners(ix, iy, base, with_mask=False)
    for k in range(4):
        irefs[k][2, 0] = idx[k]
        wrefs[k][2, 0] = wgt[k]


def _compute_idx_weights(gx, gy):
    ishape = jax.ShapeDtypeStruct((3, N, _SUBL, _LANE), jnp.int32)
    wshape = jax.ShapeDtypeStruct((3, N, _SUBL, _LANE), jnp.float32)
    in_spec = pl.BlockSpec((1, _SUBL, _LANE), lambda n: (n, 0, 0))
    out_spec = pl.BlockSpec((3, 1, _SUBL, _LANE), lambda n: (0, n, 0, 0))
    return pl.pallas_call(
        _idx_weight_body,
        grid=(N,),
        in_specs=[in_spec, in_spec],
        out_specs=[out_spec] * 8,
        out_shape=[ishape] * 4 + [wshape] * 4,
    )(gx, gy)


def _sc_body(table, i0, i1, i2, i3, w0, w1, w2, w3, out,
             iv0, iv1, iv2, iv3, wv0, wv1, wv2, wv3,
             g0, g1, g2, g3, ov, sem, semg):
    wid = lax.axis_index("s") * NC + lax.axis_index("c")
    base0 = wid * RPW

    def chunk(ci, carry):
        base = base0 + ci * B
        sl = pl.ds(base, B)
        cps = [
            pltpu.async_copy(i0.at[sl], iv0, sem),
            pltpu.async_copy(i1.at[sl], iv1, sem),
            pltpu.async_copy(i2.at[sl], iv2, sem),
            pltpu.async_copy(i3.at[sl], iv3, sem),
            pltpu.async_copy(w0.at[sl], wv0, sem),
            pltpu.async_copy(w1.at[sl], wv1, sem),
            pltpu.async_copy(w2.at[sl], wv2, sem),
            pltpu.async_copy(w3.at[sl], wv3, sem),
        ]
        for cp in cps:
            cp.wait()
        gcps = [
            pltpu.async_copy(table.at[iv0], g0, semg),
            pltpu.async_copy(table.at[iv1], g1, semg),
            pltpu.async_copy(table.at[iv2], g2, semg),
            pltpu.async_copy(table.at[iv3], g3, semg),
        ]
        for cp in gcps:
            cp.wait()

        def group(gi, c2):
            rows = gi * L + lax.iota(jnp.int32, L)
            rowoff = rows * C
            wa = wv0[pl.ds(gi * L, L)]
            wb = wv1[pl.ds(gi * L, L)]
            wc = wv2[pl.ds(gi * L, L)]
            wd = wv3[pl.ds(gi * L, L)]

            def chan(c, c3):
                colv = jnp.full((L,), c, jnp.int32)
                acc = plsc.load_gather(g0, [rows, colv]) * wa
                acc = acc + plsc.load_gather(g1, [rows, colv]) * wb
                acc = acc + plsc.load_gather(g2, [rows, colv]) * wc
                acc = acc + plsc.load_gather(g3, [rows, colv]) * wd
                plsc.store_scatter(ov, [rowoff + c], acc)
                return c3

            return lax.fori_loop(0, C, chan, c2)

        lax.fori_loop(0, GROUPS, group, 0)
        pltpu.sync_copy(ov, out.at[pl.ds(base * C, B * C)])
        return carry

    lax.fori_loop(0, NCHUNK, chunk, 0)


@functools.cache
def _sc_gather():
    return pl.kernel(
        _sc_body,
        out_type=jax.ShapeDtypeStruct((R * C,), jnp.float32),
        mesh=plsc.VectorSubcoreMesh(
            core_axis_name="c", subcore_axis_name="s",
            num_cores=NC, num_subcores=NS),
        compiler_params=pltpu.CompilerParams(needs_layout_passes=False),
        scratch_types=[pltpu.VMEM((B,), jnp.int32)] * 4
        + [pltpu.VMEM((B,), jnp.float32)] * 4
        + [pltpu.VMEM((B, CP), jnp.float32)] * 4
        + [pltpu.VMEM((B * C,), jnp.float32),
           pltpu.SemaphoreType.DMA,
           pltpu.SemaphoreType.DMA],
    )


def kernel(x, T):
    x_t = x.transpose(0, 2, 3, 1)
    x_flat = jnp.concatenate(
        [x_t, jnp.zeros((N, H, W, CP - C), x.dtype)], axis=-1,
    ).reshape(N * P, CP)
    gx = T[..., 0].reshape(N, _SUBL, _LANE)
    gy = T[..., 1].reshape(N, _SUBL, _LANE)
    i0, i1, i2, i3, w0, w1, w2, w3 = _compute_idx_weights(gx, gy)
    flat = lambda a: a.reshape(R)
    out = _sc_gather()(x_flat,
                     flat(i0), flat(i1), flat(i2), flat(i3),
                     flat(w0), flat(w1), flat(w2), flat(w3))
    y = out.reshape(3, N, H, W, C)
    return (jnp.transpose(y[0], (0, 3, 1, 2)),
            jnp.transpose(y[1], (0, 3, 1, 2)),
            jnp.transpose(y[2], (0, 3, 1, 2)))


# speedup vs baseline: 2.9454x; 2.9454x over previous
import functools

import jax
import jax.numpy as jnp
from jax import lax
from jax.experimental import pallas as pl
from jax.experimental.pallas import tpu as pltpu
from jax.experimental.pallas import tpu_sc as plsc

N, C, H, W = 4, 96, 224, 224
P = H * W
R = 3 * N * P
NC, NS, L = 2, 16, 16
NW = NC * NS
RPW = R // NW
B = 64
CP = 128
SB = 14
NSB = RPW // (SB * B)
GROUPS = B // L

_SUBL = 392
_LANE = 128


def _corners(ix, iy, base, with_mask):
    x0f = jnp.floor(ix)
    y0f = jnp.floor(iy)
    x1f = x0f + 1.0
    y1f = y0f + 1.0
    wx1 = ix - x0f
    wx0 = 1.0 - wx1
    wy1 = iy - y0f
    wy0 = 1.0 - wy1

    def clampi(v, hi):
        return jnp.clip(v, 0.0, hi).astype(jnp.int32)

    xi0 = clampi(x0f, W - 1.0)
    xi1 = clampi(x1f, W - 1.0)
    yi0 = clampi(y0f, H - 1.0)
    yi1 = clampi(y1f, H - 1.0)

    def flat(yi, xi):
        return base + yi * W + xi

    idx = (flat(yi0, xi0), flat(yi0, xi1), flat(yi1, xi0), flat(yi1, xi1))
    w = [wy0 * wx0, wy0 * wx1, wy1 * wx0, wy1 * wx1]
    if with_mask:
        def m(yf, xf):
            return ((xf >= 0.0) & (xf <= W - 1.0)
                    & (yf >= 0.0) & (yf <= H - 1.0)).astype(jnp.float32)
        w[0] = w[0] * m(y0f, x0f)
        w[1] = w[1] * m(y0f, x1f)
        w[2] = w[2] * m(y1f, x0f)
        w[3] = w[3] * m(y1f, x1f)
    return idx, tuple(w)


def _reflect(c, span):
    cc = jnp.abs(c)
    extra = jnp.mod(cc, span)
    flips = jnp.floor(cc / span)
    return jnp.where(jnp.mod(flips, 2.0) == 0.0, extra, span - extra)


def _idx_weight_body(gx_ref, gy_ref, i0, i1, i2, i3, w0, w1, w2, w3):
    n = pl.program_id(0)
    base = n * P
    gx = gx_ref[0]
    gy = gy_ref[0]
    irefs = (i0, i1, i2, i3)
    wrefs = (w0, w1, w2, w3)

    ix = (gx + 1.0) * (0.5 * (W - 1.0))
    iy = (gy + 1.0) * (0.5 * (H - 1.0))
    idx, wgt = _corners(ix, iy, base, with_mask=True)
    for k in range(4):
        irefs[k][0, 0] = idx[k]
        wrefs[k][0, 0] = wgt[k]

    ix = jnp.clip(((gx + 1.0) * W - 1.0) * 0.5, 0.0, W - 1.0)
    iy = jnp.clip(((gy + 1.0) * H - 1.0) * 0.5, 0.0, H - 1.0)
    idx, wgt = _corners(ix, iy, base, with_mask=False)
    for k in range(4):
        irefs[k][1, 0] = idx[k]
        wrefs[k][1, 0] = wgt[k]

    ix = (gx + 1.0) * (0.5 * (W - 1.0))
    iy = (gy + 1.0) * (0.5 * (H - 1.0))
    ix = jnp.clip(_reflect(ix, W - 1.0), 0.0, W - 1.0)
    iy = jnp.clip(_reflect(iy, H - 1.0), 0.0, H - 1.0)
    idx, wgt = _corners(ix, iy, base, with_mask=False)
    for k in range(4):
        irefs[k][2, 0] = idx[k]
        wrefs[k][2, 0] = wgt[k]


def _compute_idx_weights(gx, gy):
    ishape = jax.ShapeDtypeStruct((3, N, _SUBL, _LANE), jnp.int32)
    wshape = jax.ShapeDtypeStruct((3, N, _SUBL, _LANE), jnp.float32)
    in_spec = pl.BlockSpec((1, _SUBL, _LANE), lambda n: (n, 0, 0))
    out_spec = pl.BlockSpec((3, 1, _SUBL, _LANE), lambda n: (0, n, 0, 0))
    return pl.pallas_call(
        _idx_weight_body,
        grid=(N,),
        in_specs=[in_spec, in_spec],
        out_specs=[out_spec] * 8,
        out_shape=[ishape] * 4 + [wshape] * 4,
    )(gx, gy)


def _sc_body(table, i0, i1, i2, i3, w0, w1, w2, w3, out,
             si0, si1, si2, si3, sw0, sw1, sw2, sw3,
             ga0, ga1, ga2, ga3, gb0, gb1, gb2, gb3,
             ov, semst, semg0, semg1):
    wid = lax.axis_index("s") * NC + lax.axis_index("c")
    base0 = wid * RPW
    sidx = (si0, si1, si2, si3)
    swgt = (sw0, sw1, sw2, sw3)
    gbuf = ((ga0, ga1, ga2, ga3), (gb0, gb1, gb2, gb3))
    semg = (semg0, semg1)

    def fire(cc, par):
        for k in range(4):
            pltpu.async_copy(
                table.at[sidx[k].at[pl.ds(cc * B, B)]], gbuf[par][k], semg[par])

    def drain(cc, par):
        for k in range(4):
            pltpu.make_async_copy(
                table.at[sidx[k].at[pl.ds(cc * B, B)]],
                gbuf[par][k], semg[par]).wait()

    def compute(cc, par):
        gk = gbuf[par]

        def group(gi, c2):
            woff = cc * B + gi * L
            wvecs = [swgt[k][pl.ds(woff, L)] for k in range(4)]
            for rl in range(L):
                r = gi * L + rl
                ws = [wvecs[k][rl] for k in range(4)]
                for cb in range(C // L):
                    sl = pl.ds(cb * L, L)
                    acc = (gk[0][r, sl] * ws[0] + gk[1][r, sl] * ws[1]
                           + gk[2][r, sl] * ws[2] + gk[3][r, sl] * ws[3])
                    ov[pl.ds(r * C + cb * L, L)] = acc
            return c2

        lax.fori_loop(0, GROUPS, group, 0)

    def superblock(s, carry):
        rbase = s * SB * B
        ssl = pl.ds(base0 + rbase, SB * B)
        stcps = [pltpu.async_copy(src.at[ssl], dst, semst)
                 for src, dst in zip((i0, i1, i2, i3), sidx)]
        stcps += [pltpu.async_copy(src.at[ssl], dst, semst)
                  for src, dst in zip((w0, w1, w2, w3), swgt)]
        for cp in stcps:
            cp.wait()

        fire(0, 0)

        def pair(j, c2):
            cc0 = 2 * j
            fire_next = cc0 + 1
            fire(fire_next, 1)
            drain(cc0, 0)
            compute(cc0, 0)
            pltpu.sync_copy(
                ov, out.at[pl.ds((base0 + rbase + cc0 * B) * C, B * C)])
            @pl.when(j < SB // 2 - 1)
            def _():
                fire(cc0 + 2, 0)
            drain(cc0 + 1, 1)
            compute(cc0 + 1, 1)
            pltpu.sync_copy(
                ov, out.at[pl.ds((base0 + rbase + (cc0 + 1) * B) * C, B * C)])
            return c2

        lax.fori_loop(0, SB // 2, pair, 0)
        return carry

    lax.fori_loop(0, NSB, superblock, 0)


@functools.cache
def _sc_gather():
    return pl.kernel(
        _sc_body,
        out_type=jax.ShapeDtypeStruct((R * C,), jnp.float32),
        mesh=plsc.VectorSubcoreMesh(
            core_axis_name="c", subcore_axis_name="s",
            num_cores=NC, num_subcores=NS),
        compiler_params=pltpu.CompilerParams(needs_layout_passes=False),
        scratch_types=[pltpu.VMEM((SB * B,), jnp.int32)] * 4
        + [pltpu.VMEM((SB * B,), jnp.float32)] * 4
        + [pltpu.VMEM((B, CP), jnp.float32)] * 8
        + [pltpu.VMEM((B * C,), jnp.float32),
           pltpu.SemaphoreType.DMA,
           pltpu.SemaphoreType.DMA,
           pltpu.SemaphoreType.DMA],
    )


def kernel(x, T):
    x_t = x.transpose(0, 2, 3, 1)
    x_flat = jnp.concatenate(
        [x_t, jnp.zeros((N, H, W, CP - C), x.dtype)], axis=-1,
    ).reshape(N * P, CP)
    gx = T[..., 0].reshape(N, _SUBL, _LANE)
    gy = T[..., 1].reshape(N, _SUBL, _LANE)
    i0, i1, i2, i3, w0, w1, w2, w3 = _compute_idx_weights(gx, gy)
    flat = lambda a: a.reshape(R)
    out = _sc_gather()(x_flat,
                     flat(i0), flat(i1), flat(i2), flat(i3),
                     flat(w0), flat(w1), flat(w2), flat(w3))
    y = out.reshape(3, N, H, W, C)
    return (jnp.transpose(y[0], (0, 3, 1, 2)),
            jnp.transpose(y[1], (0, 3, 1, 2)),
            jnp.transpose(y[2], (0, 3, 1, 2)))
